# baseline (speedup 1.0000x reference)
import jax
import jax.numpy as jnp
from jax.experimental import pallas as pl

N = 10000
K = 64


def kernel(inputs):
    vals, inds = jax.lax.top_k(inputs, K + 1)
    weights = jnp.reshape(vals, (-1,))
    targets = jnp.reshape(inds, (-1,))
    sources = jnp.repeat(jnp.arange(0, N, dtype=jnp.int32), K + 1)
    return (sources, targets, weights)


# speedup vs baseline: 12.8348x; 12.8348x over previous
import dataclasses
import functools

import jax
import jax.numpy as jnp
from jax import lax
from jax.experimental import pallas as pl
from jax.experimental.pallas import tpu as pltpu
from jax.experimental.pallas import tpu_sc as plsc

N = 10000
KP1 = 65
OUTW = 80
NGROUP = 40
ROW_PAD = NGROUP * 256
NEG = float("-inf")
NW = 32


def _topk_sc(x):
    mesh = plsc.VectorSubcoreMesh(core_axis_name="c", subcore_axis_name="s")
    cp = pltpu.CompilerParams()
    if "needs_layout_passes" in pltpu.CompilerParams.__dataclass_fields__:
        cp = dataclasses.replace(cp, needs_layout_passes=False)

    @functools.partial(
        pl.kernel,
        out_type=(
            jax.ShapeDtypeStruct((N, OUTW), jnp.int32),
            jax.ShapeDtypeStruct((N, OUTW), jnp.int32),
            jax.ShapeDtypeStruct((N, OUTW), jnp.float32),
        ),
        mesh=mesh,
        compiler_params=cp,
        scratch_types=[
            pltpu.VMEM((ROW_PAD,), jnp.float32),
            pltpu.VMEM((NGROUP, 16), jnp.float32),
            pltpu.VMEM((48,), jnp.float32),
            pltpu.VMEM((OUTW,), jnp.float32),
            pltpu.VMEM((OUTW,), jnp.int32),
            pltpu.VMEM((OUTW,), jnp.int32),
        ],
    )
    def k(xf_hbm, src_hbm, tgt_hbm, w_hbm, row_v, m_v, m2_v, wv_v, wi_v, ws_v):
        wid = lax.axis_index("s") * 2 + lax.axis_index("c")
        nrows = jnp.where(wid < 16, 313, 312)
        base = wid * 312 + jnp.minimum(wid, 16)
        iota = lax.iota(jnp.int32, 16)
        neg16 = jnp.full((16,), NEG, jnp.float32)
        zero16 = jnp.zeros((16,), jnp.int32)
        lane0 = iota == 0

        def store1(ref, idxs, val):
            plsc.store_scatter(ref, [zero16 + i for i in idxs],
                               jnp.zeros((16,), val.dtype) + val, mask=lane0)

        for t in range(15):
            row_v[pl.ds(N + 16 * t, 16)] = neg16
        m2_v[pl.ds(32, 16)] = neg16
        for t in range(OUTW // 16):
            wv_v[pl.ds(16 * t, 16)] = jnp.zeros((16,), jnp.float32)
            wi_v[pl.ds(16 * t, 16)] = zero16

        @pl.loop(0, nrows)
        def _row(i):
            r = base + i
            pltpu.sync_copy(xf_hbm.at[pl.ds(r * N, N)], row_v.at[pl.ds(0, N)])

            @pl.loop(0, NGROUP)
            def _grp(g):
                b = g * 256
                m = row_v[pl.ds(b, 16)]
                for j in range(1, 16):
                    m = jnp.maximum(m, row_v[pl.ds(b + 16 * j, 16)])
                m_v[g] = m
                store1(m2_v, [g], jnp.max(m))

            @pl.loop(0, KP1)
            def _ext(t):
                a0 = m2_v[pl.ds(0, 16)]
                a1 = m2_v[pl.ds(16, 16)]
                a2 = m2_v[pl.ds(32, 16)]
                gmax = jnp.max(jnp.maximum(jnp.maximum(a0, a1), a2))
                e0 = a0 == gmax
                e1 = a1 == gmax
                e2 = a2 == gmax
                f0 = jnp.max(plsc.all_reduce_ffs(e0))
                f1 = jnp.max(plsc.all_reduce_ffs(e1))
                f2 = jnp.max(plsc.all_reduce_ffs(e2))
                g_ = jnp.where(f0 < 16, f0,
                               jnp.where(f1 < 16, 16 + f1, 32 + f2))
                mv = m_v[g_]
                l_ = jnp.max(plsc.all_reduce_ffs(mv == gmax))
                cbase = g_ * 256 + l_
                idxv = cbase + iota * 16
                cv = plsc.load_gather(row_v, [idxv])
                j_ = jnp.max(plsc.all_reduce_ffs(cv == gmax))
                col = cbase + 16 * j_
                store1(row_v, [col], jnp.float32(NEG))
                cv2 = jnp.where(iota == j_, neg16, cv)
                nm = jnp.max(cv2)
                store1(m_v, [g_, l_], nm)
                store1(m2_v, [g_], jnp.max(jnp.where(iota == l_, nm, mv)))
                store1(wv_v, [t], gmax)
                store1(wi_v, [t], col)

            for t in range(OUTW // 16):
                ws_v[pl.ds(16 * t, 16)] = zero16 + r
            pltpu.sync_copy(wv_v, w_hbm.at[r])
            pltpu.sync_copy(wi_v, tgt_hbm.at[r])
            pltpu.sync_copy(ws_v, src_hbm.at[r])

    return k(jnp.reshape(x, (-1,)))


def kernel(inputs):
    src, tgt, w = _topk_sc(inputs)
    sources = src[:, :KP1].reshape(-1)
    targets = tgt[:, :KP1].reshape(-1)
    weights = w[:, :KP1].reshape(-1)
    return (sources, targets, weights)


# speedup vs baseline: 22.9112x; 1.7851x over previous
import dataclasses
import functools

import jax
import jax.numpy as jnp
from jax import lax
from jax.experimental import pallas as pl
from jax.experimental.pallas import tpu as pltpu
from jax.experimental.pallas import tpu_sc as plsc

N = 10000
KP1 = 65
OUTW = 80
NGROUP = 40
ROW_PAD = NGROUP * 256
NEG = float("-inf")
NW = 32
R = 4


def _topk_sc(x):
    mesh = plsc.VectorSubcoreMesh(core_axis_name="c", subcore_axis_name="s")
    cp = pltpu.CompilerParams()
    if "needs_layout_passes" in pltpu.CompilerParams.__dataclass_fields__:
        cp = dataclasses.replace(cp, needs_layout_passes=False)

    @functools.partial(
        pl.kernel,
        out_type=(
            jax.ShapeDtypeStruct((N * OUTW,), jnp.int32),
            jax.ShapeDtypeStruct((N * OUTW,), jnp.int32),
            jax.ShapeDtypeStruct((N * OUTW,), jnp.float32),
        ),
        mesh=mesh,
        compiler_params=cp,
        scratch_types=[
            pltpu.VMEM((R * ROW_PAD,), jnp.float32),
            pltpu.VMEM((R * NGROUP * 16,), jnp.float32),
            pltpu.VMEM((R * 48,), jnp.float32),
            pltpu.VMEM((R * OUTW,), jnp.float32),
            pltpu.VMEM((R * OUTW,), jnp.int32),
            pltpu.VMEM((R * OUTW,), jnp.int32),
        ],
    )
    def k(xf_hbm, src_hbm, tgt_hbm, w_hbm, rows_v, m_v, m2_v, wv_v, wi_v,
          ws_v):
        wid = lax.axis_index("s") * 2 + lax.axis_index("c")
        s = (N * wid) // NW
        e = (N * (wid + 1)) // NW
        nb = (e - s + R - 1) // R
        iota = lax.iota(jnp.int32, 16)
        neg16 = jnp.full((16,), NEG, jnp.float32)
        zero16 = jnp.zeros((16,), jnp.int32)
        lane0 = iota == 0

        def store1(ref, idxs, val):
            plsc.store_scatter(ref, [zero16 + i for i in idxs],
                               jnp.zeros((16,), jnp.result_type(val)) + val,
                               mask=lane0)

        for q in range(R):
            for t in range(15):
                rows_v[pl.ds(q * ROW_PAD + N + 16 * t, 16)] = neg16
            m2_v[pl.ds(q * 48 + 32, 16)] = neg16
            for t in range(OUTW // 16):
                wv_v[pl.ds(q * OUTW + 16 * t, 16)] = jnp.zeros((16,),
                                                               jnp.float32)
                wi_v[pl.ds(q * OUTW + 16 * t, 16)] = zero16

        @pl.loop(0, nb)
        def _blk(b):
            rs = jnp.minimum(s + R * b, e - R)
            for q in range(R):
                pltpu.sync_copy(xf_hbm.at[pl.ds((rs + q) * N, N)],
                                rows_v.at[pl.ds(q * ROW_PAD, N)])

            @pl.loop(0, NGROUP)
            def _grp(g):
                bofs = g * 256
                for q in range(R):
                    m = rows_v[pl.ds(q * ROW_PAD + bofs, 16)]
                    for j in range(1, 16):
                        m = jnp.maximum(
                            m, rows_v[pl.ds(q * ROW_PAD + bofs + 16 * j, 16)])
                    m_v[pl.ds(q * 640 + g * 16, 16)] = m
                    store1(m2_v, [q * 48 + g], jnp.max(m))

            carry0 = tuple(m2_v[pl.ds(q * 48 + 16 * i, 16)]
                           for q in range(R) for i in range(3))

            @pl.loop(0, KP1, init_carry=carry0)
            def _ext(t, carry):
                cs = list(carry)
                for q in range(R):
                    c0, c1, c2 = cs[3 * q:3 * q + 3]
                    gmax = jnp.max(jnp.maximum(jnp.maximum(c0, c1), c2))
                    f0 = plsc.all_reduce_ffs(c0 == gmax)
                    f1 = plsc.all_reduce_ffs(c1 == gmax)
                    f2 = plsc.all_reduce_ffs(c2 == gmax)
                    g_ = jnp.where(f0 < 16, f0,
                                   jnp.where(f1 < 16, 16 + f1, 32 + f2))
                    mv = plsc.load_gather(m_v, [q * 640 + g_ * 16 + iota])
                    l_ = plsc.all_reduce_ffs(mv == gmax)
                    cbase = g_ * 256 + l_
                    cv = plsc.load_gather(
                        rows_v, [q * ROW_PAD + cbase + iota * 16])
                    j_ = plsc.all_reduce_ffs(cv == gmax)
                    col = cbase + 16 * j_
                    store1(rows_v, [q * ROW_PAD + col], jnp.float32(NEG))
                    cv2 = jnp.where(iota == j_, neg16, cv)
                    nm = jnp.max(cv2)
                    store1(m_v, [q * 640 + g_ * 16 + l_], nm)
                    mv2 = jnp.where(iota == l_, nm, mv)
                    nm2 = jnp.max(mv2)
                    cs[3 * q + 0] = jnp.where(iota == g_, nm2, c0)
                    cs[3 * q + 1] = jnp.where(iota + 16 == g_, nm2, c1)
                    cs[3 * q + 2] = jnp.where(iota + 32 == g_, nm2, c2)
                    store1(wv_v, [q * OUTW + t], gmax)
                    store1(wi_v, [q * OUTW + t], col)
                return tuple(cs)

            for q in range(R):
                r = rs + q
                for t in range(OUTW // 16):
                    ws_v[pl.ds(q * OUTW + 16 * t, 16)] = zero16 + r
                pltpu.sync_copy(wv_v.at[pl.ds(q * OUTW, OUTW)],
                                w_hbm.at[pl.ds(r * OUTW, OUTW)])
                pltpu.sync_copy(wi_v.at[pl.ds(q * OUTW, OUTW)],
                                tgt_hbm.at[pl.ds(r * OUTW, OUTW)])
                pltpu.sync_copy(ws_v.at[pl.ds(q * OUTW, OUTW)],
                                src_hbm.at[pl.ds(r * OUTW, OUTW)])

    return k(jnp.reshape(x, (-1,)))


def kernel(inputs):
    src, tgt, w = _topk_sc(inputs)
    sources = src.reshape(N, OUTW)[:, :KP1].reshape(-1)
    targets = tgt.reshape(N, OUTW)[:, :KP1].reshape(-1)
    weights = w.reshape(N, OUTW)[:, :KP1].reshape(-1)
    return (sources, targets, weights)


# speedup vs baseline: 29.5874x; 1.2914x over previous
import dataclasses
import functools

import jax
import jax.numpy as jnp
from jax import lax
from jax.experimental import pallas as pl
from jax.experimental.pallas import tpu as pltpu
from jax.experimental.pallas import tpu_sc as plsc

N = 10000
KP1 = 65
OUTW = 80
NGROUP = 40
ROW_PAD = NGROUP * 256
NEG = float("-inf")
NW = 32
R = 4


def _topk_sc(x):
    mesh = plsc.VectorSubcoreMesh(core_axis_name="c", subcore_axis_name="s")
    cp = pltpu.CompilerParams()
    if "needs_layout_passes" in pltpu.CompilerParams.__dataclass_fields__:
        cp = dataclasses.replace(cp, needs_layout_passes=False)

    @functools.partial(
        pl.kernel,
        out_type=(
            jax.ShapeDtypeStruct((N * OUTW,), jnp.int32),
            jax.ShapeDtypeStruct((N * OUTW,), jnp.int32),
            jax.ShapeDtypeStruct((N * OUTW,), jnp.float32),
        ),
        mesh=mesh,
        compiler_params=cp,
        scratch_types=[
            pltpu.VMEM((R * ROW_PAD,), jnp.float32),
            pltpu.VMEM((R * ROW_PAD,), jnp.float32),
            pltpu.VMEM((R * NGROUP * 16,), jnp.float32),
            pltpu.VMEM((R * 48,), jnp.float32),
            pltpu.VMEM((R * OUTW,), jnp.float32),
            pltpu.VMEM((R * OUTW,), jnp.int32),
            pltpu.VMEM((R * OUTW,), jnp.int32),
            pltpu.SemaphoreType.DMA,
            pltpu.SemaphoreType.DMA,
            pltpu.SemaphoreType.DMA,
        ],
    )
    def k(xf_hbm, src_hbm, tgt_hbm, w_hbm, rows0_v, rows1_v, m_v, m2_v,
          wv_v, wi_v, ws_v, isem0, isem1, osem):
        rowsets = (rows0_v, rows1_v)
        isems = (isem0, isem1)
        wid = lax.axis_index("s") * 2 + lax.axis_index("c")
        s = (N * wid) // NW
        e = (N * (wid + 1)) // NW
        nb = (e - s + R - 1) // R
        iota = lax.iota(jnp.int32, 16)
        neg16 = jnp.full((16,), NEG, jnp.float32)
        zero16 = jnp.zeros((16,), jnp.int32)
        lane0 = iota == 0

        def store1(ref, idxs, val):
            plsc.store_scatter(ref, [zero16 + i for i in idxs],
                               jnp.zeros((16,), jnp.result_type(val)) + val,
                               mask=lane0)

        for q in range(R):
            for t in range(15):
                rows0_v[pl.ds(q * ROW_PAD + N + 16 * t, 16)] = neg16
                rows1_v[pl.ds(q * ROW_PAD + N + 16 * t, 16)] = neg16
            m2_v[pl.ds(q * 48 + 32, 16)] = neg16
            for t in range(OUTW // 16):
                wv_v[pl.ds(q * OUTW + 16 * t, 16)] = jnp.zeros((16,),
                                                               jnp.float32)
                wi_v[pl.ds(q * OUTW + 16 * t, 16)] = zero16

        def block_start(b):
            return jnp.minimum(s + R * b, e - R)

        def issue_in(b, si):
            rs = block_start(b)
            for q in range(R):
                pltpu.async_copy(xf_hbm.at[pl.ds((rs + q) * N, N)],
                                 rowsets[si].at[pl.ds(q * ROW_PAD, N)],
                                 isems[si])

        def wait_in(si):
            for q in range(R):
                pltpu.make_async_copy(
                    xf_hbm.at[pl.ds(0, N)],
                    rowsets[si].at[pl.ds(q * ROW_PAD, N)],
                    isems[si]).wait()

        def compute(b, si):
            rows_v = rowsets[si]
            rs = block_start(b)

            @pl.loop(0, NGROUP)
            def _grp(g):
                bofs = g * 256
                for q in range(R):
                    m = rows_v[pl.ds(q * ROW_PAD + bofs, 16)]
                    for j in range(1, 16):
                        m = jnp.maximum(
                            m, rows_v[pl.ds(q * ROW_PAD + bofs + 16 * j, 16)])
                    m_v[pl.ds(q * 640 + g * 16, 16)] = m
                    store1(m2_v, [q * 48 + g], jnp.max(m))

            carry0 = tuple(m2_v[pl.ds(q * 48 + 16 * i, 16)]
                           for q in range(R) for i in range(3))

            @pl.loop(0, KP1, init_carry=carry0)
            def _ext(t, carry):
                cs = list(carry)
                for q in range(R):
                    c0, c1, c2 = cs[3 * q:3 * q + 3]
                    gmax = jnp.max(jnp.maximum(jnp.maximum(c0, c1), c2))
                    f0 = plsc.all_reduce_ffs(c0 == gmax)
                    f1 = plsc.all_reduce_ffs(c1 == gmax)
                    f2 = plsc.all_reduce_ffs(c2 == gmax)
                    g_ = jnp.where(f0 < 16, f0,
                                   jnp.where(f1 < 16, 16 + f1, 32 + f2))
                    mv = plsc.load_gather(m_v, [q * 640 + g_ * 16 + iota])
                    l_ = plsc.all_reduce_ffs(mv == gmax)
                    cbase = g_ * 256 + l_
                    cv = plsc.load_gather(
                        rows_v, [q * ROW_PAD + cbase + iota * 16])
                    j_ = plsc.all_reduce_ffs(cv == gmax)
                    col = cbase + 16 * j_
                    store1(rows_v, [q * ROW_PAD + col], jnp.float32(NEG))
                    cv2 = jnp.where(iota == j_, neg16, cv)
                    nm = jnp.max(cv2)
                    store1(m_v, [q * 640 + g_ * 16 + l_], nm)
                    mv2 = jnp.where(iota == l_, nm, mv)
                    nm2 = jnp.max(mv2)
                    cs[3 * q + 0] = jnp.where(iota == g_, nm2, c0)
                    cs[3 * q + 1] = jnp.where(iota + 16 == g_, nm2, c1)
                    cs[3 * q + 2] = jnp.where(iota + 32 == g_, nm2, c2)
                    store1(wv_v, [q * OUTW + t], gmax)
                    store1(wi_v, [q * OUTW + t], col)
                return tuple(cs)

            hs = []
            for q in range(R):
                r = rs + q
                for t in range(OUTW // 16):
                    ws_v[pl.ds(q * OUTW + 16 * t, 16)] = zero16 + r
                hs.append(pltpu.async_copy(
                    wv_v.at[pl.ds(q * OUTW, OUTW)],
                    w_hbm.at[pl.ds(r * OUTW, OUTW)], osem))
                hs.append(pltpu.async_copy(
                    wi_v.at[pl.ds(q * OUTW, OUTW)],
                    tgt_hbm.at[pl.ds(r * OUTW, OUTW)], osem))
                hs.append(pltpu.async_copy(
                    ws_v.at[pl.ds(q * OUTW, OUTW)],
                    src_hbm.at[pl.ds(r * OUTW, OUTW)], osem))
            for h in hs:
                h.wait()

        NB = 80
        issue_in(0, 0)

        @pl.loop(0, NB // 2)
        def _h(h):
            b0 = 2 * h
            wait_in(0)
            issue_in(b0 + 1, 1)
            compute(b0, 0)
            wait_in(1)
            issue_in(b0 + 2, 0)
            compute(b0 + 1, 1)

    return k(jnp.reshape(x, (-1,)))


def kernel(inputs):
    src, tgt, w = _topk_sc(inputs)
    sources = src.reshape(N, OUTW)[:, :KP1].reshape(-1)
    targets = tgt.reshape(N, OUTW)[:, :KP1].reshape(-1)
    weights = w.reshape(N, OUTW)[:, :KP1].reshape(-1)
    return (sources, targets, weights)
